# baseline (speedup 1.0000x reference)
import functools

import jax
import jax.numpy as jnp
from jax import lax
from jax.experimental import pallas as pl
from jax.experimental.pallas import tpu as pltpu
from jax.experimental.pallas import tpu_sc as plsc

_B = 32
_N = 2048
_C = 1000
_M = 128

_NC = 2
_NS = 16
_LANES = 16
_NG = _M // _LANES
_CT = _C // 8
_TPC = 25
_NCHUNK = _CT // _TPC


def _sc_body(cls_hbm, labels_hbm, sumexp_hbm, picked_hbm,
             buf0, buf1, labels_v, stage_sum, stage_pick,
             sem0, sem1, sem_l):
    wid = lax.axis_index("s") * _NC + lax.axis_index("c")

    lcp = pltpu.async_copy(labels_hbm.at[wid], labels_v, sem_l)

    bufs = (buf0, buf1)
    sems = (sem0, sem1)

    def issue(chunk):
        c0 = chunk * _TPC * 8
        return [
            pltpu.async_copy(
                cls_hbm.at[wid, pl.ds(c0 + t * 8, 8), pl.ds(0, _M)],
                bufs[chunk % 2].at[t], sems[chunk % 2])
            for t in range(_TPC)
        ]

    pending = {0: issue(0)}
    il = lax.iota(jnp.int32, _LANES)
    zero = jnp.zeros((_LANES,), jnp.float32)
    for k in range(_NG):
        stage_sum[pl.ds(k * _LANES, _LANES)] = zero

    lcp.wait()
    labels16s = [labels_v[pl.ds(k * _LANES, _LANES)] for k in range(_NG)]
    cts = [lab >> 3 for lab in labels16s]
    c8s = [lab & 7 for lab in labels16s]

    pick = [zero for _ in range(_NG)]

    for chunk in range(_NCHUNK):
        if chunk + 1 < _NCHUNK:
            pending[chunk + 1] = issue(chunk + 1)
        for d in pending.pop(chunk):
            d.wait()
        buf = bufs[chunk % 2]

        def _tile(t, carry):
            for k in range(_NG):
                e = [jnp.exp(buf[t, c8, pl.ds(k * _LANES, _LANES)])
                     for c8 in range(8)]
                s = ((e[0] + e[1]) + (e[2] + e[3])) + (
                    (e[4] + e[5]) + (e[6] + e[7]))
                plsc.addupdate(stage_sum.at[pl.ds(k * _LANES, _LANES)], s)
            return carry

        lax.fori_loop(0, _TPC, _tile, 0)

        for k in range(_NG):
            t_rel = cts[k] - chunk * _TPC
            inb = (t_rel >= 0) & (t_rel < _TPC)
            t_safe = jnp.clip(t_rel, 0, _TPC - 1)
            g = plsc.load_gather(buf, [t_safe, c8s[k], k * _LANES + il])
            pick[k] = jnp.where(inb, g, pick[k])

    for k in range(_NG):
        stage_pick[pl.ds(k * _LANES, _LANES)] = pick[k]

    pltpu.sync_copy(stage_sum, sumexp_hbm.at[wid])
    pltpu.sync_copy(stage_pick, picked_hbm.at[wid])


_sc_call = functools.partial(
    pl.kernel,
    out_type=[
        jax.ShapeDtypeStruct((_B, _M), jnp.float32),
        jax.ShapeDtypeStruct((_B, _M), jnp.float32),
    ],
    mesh=plsc.VectorSubcoreMesh(
        core_axis_name="c", subcore_axis_name="s",
        num_cores=_NC, num_subcores=_NS),
    compiler_params=pltpu.CompilerParams(needs_layout_passes=False),
    scratch_types=[
        pltpu.VMEM((_TPC, 8, _M), jnp.float32),
        pltpu.VMEM((_TPC, 8, _M), jnp.float32),
        pltpu.VMEM((_M,), jnp.int32),
        pltpu.VMEM((_M,), jnp.float32),
        pltpu.VMEM((_M,), jnp.float32),
        pltpu.SemaphoreType.DMA,
        pltpu.SemaphoreType.DMA,
        pltpu.SemaphoreType.DMA,
    ],
)(_sc_body)


def _tc_pre_body(obj_ref, pbox_ref, tbox_ref, out_ref):
    x = obj_ref[...]
    col = lax.broadcasted_iota(jnp.int32, (_B, _N), 1)
    t = (col < _M).astype(jnp.float32)
    bce = jnp.maximum(x, 0.0) - x * t + jnp.log1p(jnp.exp(-jnp.abs(x)))
    out_ref[0] = jnp.sum(bce) * (1.0 / (_B * _N))
    out_ref[1] = jnp.sum(jnp.abs(pbox_ref[...] - tbox_ref[...])) * (
        1.0 / (_B * _M * 4))


def _tc_fin_body(pre_ref, sumexp_ref, picked_ref, o0, o1, o2, o3):
    lse = jnp.log(sumexp_ref[...])
    class_loss = jnp.sum(lse - picked_ref[...]) * (1.0 / (_B * _M))
    obj_loss = pre_ref[0]
    box_loss = pre_ref[1]
    o0[0] = box_loss + obj_loss + class_loss
    o1[0] = box_loss
    o2[0] = obj_loss
    o3[0] = class_loss


def kernel(pred_boxes, pred_obj, pred_class, tgt_boxes, tgt_labels):
    labels = tgt_labels.astype(jnp.int32)
    cls_t = jnp.swapaxes(pred_class, 1, 2)
    sumexp, picked = _sc_call(cls_t, labels)

    pbt = jnp.swapaxes(pred_boxes, 1, 2)
    tbt = jnp.swapaxes(tgt_boxes, 1, 2)

    pre = pl.pallas_call(
        _tc_pre_body,
        out_shape=jax.ShapeDtypeStruct((2,), jnp.float32),
        grid=(1,),
        in_specs=[
            pl.BlockSpec((_B, _N), lambda i: (0, 0)),
            pl.BlockSpec((_B, 4, _M), lambda i: (0, 0, 0)),
            pl.BlockSpec((_B, 4, _M), lambda i: (0, 0, 0)),
        ],
        out_specs=pl.BlockSpec(memory_space=pltpu.SMEM),
    )(pred_obj, pbt, tbt)

    scalar = jax.ShapeDtypeStruct((1,), jnp.float32)
    smem = pl.BlockSpec(memory_space=pltpu.SMEM)
    o0, o1, o2, o3 = pl.pallas_call(
        _tc_fin_body,
        out_shape=[scalar, scalar, scalar, scalar],
        grid=(1,),
        in_specs=[
            smem,
            pl.BlockSpec((_B, _M), lambda i: (0, 0)),
            pl.BlockSpec((_B, _M), lambda i: (0, 0)),
        ],
        out_specs=[smem, smem, smem, smem],
    )(pre, sumexp, picked)
    return (o0.reshape(()), o1.reshape(()), o2.reshape(()), o3.reshape(()))


# speedup vs baseline: 1.0689x; 1.0689x over previous
import functools

import jax
import jax.numpy as jnp
from jax import lax
from jax.experimental import pallas as pl
from jax.experimental.pallas import tpu as pltpu
from jax.experimental.pallas import tpu_sc as plsc

_B = 32
_N = 2048
_C = 1000
_M = 128

_NC = 2
_NS = 16
_LANES = 16
_NG = _M // _LANES
_CT = _C // 8
_TPC = 25
_NCHUNK = _CT // _TPC


def _sc_body(cls_hbm, labels_hbm, sumexp_hbm, picked_hbm,
             buf0, buf1, labels_v, stage_sum, stage_pick,
             sem0, sem1, sem_l):
    wid = lax.axis_index("s") * _NC + lax.axis_index("c")

    lcp = pltpu.async_copy(labels_hbm.at[wid], labels_v, sem_l)

    bufs = (buf0, buf1)
    sems = (sem0, sem1)

    def issue(chunk):
        c0 = chunk * _TPC * 8
        return [
            pltpu.async_copy(
                cls_hbm.at[wid, pl.ds(c0 + t * 8, 8), pl.ds(0, _M)],
                bufs[chunk % 2].at[t], sems[chunk % 2])
            for t in range(_TPC)
        ]

    pending = {0: issue(0)}
    il = lax.iota(jnp.int32, _LANES)
    zero = jnp.zeros((_LANES,), jnp.float32)

    lcp.wait()
    labels16s = [labels_v[pl.ds(k * _LANES, _LANES)] for k in range(_NG)]
    cts = [lab >> 3 for lab in labels16s]
    c8s = [lab & 7 for lab in labels16s]

    pick = [zero for _ in range(_NG)]
    acc = [[zero, zero] for _ in range(_NG)]

    for chunk in range(_NCHUNK):
        if chunk + 1 < _NCHUNK:
            pending[chunk + 1] = issue(chunk + 1)
        for d in pending.pop(chunk):
            d.wait()
        buf = bufs[chunk % 2]

        def _tile(t, carry):
            a = [[carry[k][p] for p in range(2)] for k in range(_NG)]
            for c8 in range(8):
                for k in range(_NG):
                    a[k][c8 % 2] = a[k][c8 % 2] + jnp.exp(
                        buf[t, c8, pl.ds(k * _LANES, _LANES)])
            return [[a[k][0], a[k][1]] for k in range(_NG)]

        acc = plsc.parallel_loop(0, _TPC, 1, unroll=2, carry=acc)(_tile)

        for k in range(_NG):
            t_rel = cts[k] - chunk * _TPC
            inb = (t_rel >= 0) & (t_rel < _TPC)
            t_safe = jnp.clip(t_rel, 0, _TPC - 1)
            g = plsc.load_gather(buf, [t_safe, c8s[k], k * _LANES + il])
            pick[k] = jnp.where(inb, g, pick[k])

    for k in range(_NG):
        stage_sum[pl.ds(k * _LANES, _LANES)] = acc[k][0] + acc[k][1]
        stage_pick[pl.ds(k * _LANES, _LANES)] = pick[k]

    pltpu.sync_copy(stage_sum, sumexp_hbm.at[wid])
    pltpu.sync_copy(stage_pick, picked_hbm.at[wid])


_sc_call = functools.partial(
    pl.kernel,
    out_type=[
        jax.ShapeDtypeStruct((_B, _M), jnp.float32),
        jax.ShapeDtypeStruct((_B, _M), jnp.float32),
    ],
    mesh=plsc.VectorSubcoreMesh(
        core_axis_name="c", subcore_axis_name="s",
        num_cores=_NC, num_subcores=_NS),
    compiler_params=pltpu.CompilerParams(needs_layout_passes=False),
    scratch_types=[
        pltpu.VMEM((_TPC, 8, _M), jnp.float32),
        pltpu.VMEM((_TPC, 8, _M), jnp.float32),
        pltpu.VMEM((_M,), jnp.int32),
        pltpu.VMEM((_M,), jnp.float32),
        pltpu.VMEM((_M,), jnp.float32),
        pltpu.SemaphoreType.DMA,
        pltpu.SemaphoreType.DMA,
        pltpu.SemaphoreType.DMA,
    ],
)(_sc_body)


def _tc_pre_body(obj_ref, pbox_ref, tbox_ref, out_ref):
    x = obj_ref[...]
    col = lax.broadcasted_iota(jnp.int32, (_B, _N), 1)
    t = (col < _M).astype(jnp.float32)
    bce = jnp.maximum(x, 0.0) - x * t + jnp.log1p(jnp.exp(-jnp.abs(x)))
    out_ref[0] = jnp.sum(bce) * (1.0 / (_B * _N))
    out_ref[1] = jnp.sum(jnp.abs(pbox_ref[...] - tbox_ref[...])) * (
        1.0 / (_B * _M * 4))


def _tc_fin_body(pre_ref, sumexp_ref, picked_ref, o0, o1, o2, o3):
    lse = jnp.log(sumexp_ref[...])
    class_loss = jnp.sum(lse - picked_ref[...]) * (1.0 / (_B * _M))
    obj_loss = pre_ref[0]
    box_loss = pre_ref[1]
    o0[0] = box_loss + obj_loss + class_loss
    o1[0] = box_loss
    o2[0] = obj_loss
    o3[0] = class_loss


def kernel(pred_boxes, pred_obj, pred_class, tgt_boxes, tgt_labels):
    labels = tgt_labels.astype(jnp.int32)
    cls_t = jnp.swapaxes(pred_class, 1, 2)
    sumexp, picked = _sc_call(cls_t, labels)

    pbt = jnp.swapaxes(pred_boxes, 1, 2)
    tbt = jnp.swapaxes(tgt_boxes, 1, 2)

    pre = pl.pallas_call(
        _tc_pre_body,
        out_shape=jax.ShapeDtypeStruct((2,), jnp.float32),
        grid=(1,),
        in_specs=[
            pl.BlockSpec((_B, _N), lambda i: (0, 0)),
            pl.BlockSpec((_B, 4, _M), lambda i: (0, 0, 0)),
            pl.BlockSpec((_B, 4, _M), lambda i: (0, 0, 0)),
        ],
        out_specs=pl.BlockSpec(memory_space=pltpu.SMEM),
    )(pred_obj, pbt, tbt)

    scalar = jax.ShapeDtypeStruct((1,), jnp.float32)
    smem = pl.BlockSpec(memory_space=pltpu.SMEM)
    o0, o1, o2, o3 = pl.pallas_call(
        _tc_fin_body,
        out_shape=[scalar, scalar, scalar, scalar],
        grid=(1,),
        in_specs=[
            smem,
            pl.BlockSpec((_B, _M), lambda i: (0, 0)),
            pl.BlockSpec((_B, _M), lambda i: (0, 0)),
        ],
        out_specs=[smem, smem, smem, smem],
    )(pre, sumexp, picked)
    return (o0.reshape(()), o1.reshape(()), o2.reshape(()), o3.reshape(()))


# speedup vs baseline: 1.3650x; 1.2770x over previous
import functools

import jax
import jax.numpy as jnp
from jax import lax
from jax.experimental import pallas as pl
from jax.experimental.pallas import tpu as pltpu
from jax.experimental.pallas import tpu_sc as plsc

_B = 32
_N = 2048
_C = 1000
_M = 128

_NC = 2
_NS = 16
_LANES = 16
_NG = _M // _LANES
_CT = _C // 8
_TPC = 25
_NCHUNK = _CT // _TPC


def _sc_body(cls_hbm, labels_hbm, sumexp_hbm, picked_hbm,
             buf0, buf1, labels_v, stage_sum, stage_pick,
             sem0, sem1, sem_l):
    wid = lax.axis_index("s") * _NC + lax.axis_index("c")

    lcp = pltpu.async_copy(labels_hbm.at[wid], labels_v, sem_l)

    bufs = (buf0, buf1)
    sems = (sem0, sem1)

    def issue(chunk):
        c0 = chunk * _TPC * 8
        return [
            pltpu.async_copy(
                cls_hbm.at[wid, pl.ds(c0 + t * 8, 8), pl.ds(0, _M)],
                bufs[chunk % 2].at[t], sems[chunk % 2])
            for t in range(_TPC)
        ]

    pending = {0: issue(0)}
    il = lax.iota(jnp.int32, _LANES)
    zero = jnp.zeros((_LANES,), jnp.float32)

    lcp.wait()
    labels16s = [labels_v[pl.ds(k * _LANES, _LANES)] for k in range(_NG)]
    cts = [lab >> 3 for lab in labels16s]
    c8s = [lab & 7 for lab in labels16s]

    pick = [zero for _ in range(_NG)]
    acc = [[zero, zero] for _ in range(_NG)]

    for chunk in range(_NCHUNK):
        if chunk + 1 < _NCHUNK:
            pending[chunk + 1] = issue(chunk + 1)
        for d in pending.pop(chunk):
            d.wait()
        buf = bufs[chunk % 2]

        def _tile(t, carry):
            a = [[carry[k][p] for p in range(2)] for k in range(_NG)]
            for c8 in range(8):
                for k in range(_NG):
                    a[k][c8 % 2] = a[k][c8 % 2] + jnp.exp(
                        buf[t, c8, pl.ds(k * _LANES, _LANES)])
            return [[a[k][0], a[k][1]] for k in range(_NG)]

        acc = lax.fori_loop(0, _TPC, _tile, acc)

        for k in range(_NG):
            t_rel = cts[k] - chunk * _TPC
            inb = (t_rel >= 0) & (t_rel < _TPC)
            t_safe = jnp.clip(t_rel, 0, _TPC - 1)
            g = plsc.load_gather(buf, [t_safe, c8s[k], k * _LANES + il])
            pick[k] = jnp.where(inb, g, pick[k])

    for k in range(_NG):
        stage_sum[pl.ds(k * _LANES, _LANES)] = acc[k][0] + acc[k][1]
        stage_pick[pl.ds(k * _LANES, _LANES)] = pick[k]

    pltpu.sync_copy(stage_sum, sumexp_hbm.at[wid])
    pltpu.sync_copy(stage_pick, picked_hbm.at[wid])


_sc_call = functools.partial(
    pl.kernel,
    out_type=[
        jax.ShapeDtypeStruct((_B, _M), jnp.float32),
        jax.ShapeDtypeStruct((_B, _M), jnp.float32),
    ],
    mesh=plsc.VectorSubcoreMesh(
        core_axis_name="c", subcore_axis_name="s",
        num_cores=_NC, num_subcores=_NS),
    compiler_params=pltpu.CompilerParams(needs_layout_passes=False),
    scratch_types=[
        pltpu.VMEM((_TPC, 8, _M), jnp.float32),
        pltpu.VMEM((_TPC, 8, _M), jnp.float32),
        pltpu.VMEM((_M,), jnp.int32),
        pltpu.VMEM((_M,), jnp.float32),
        pltpu.VMEM((_M,), jnp.float32),
        pltpu.SemaphoreType.DMA,
        pltpu.SemaphoreType.DMA,
        pltpu.SemaphoreType.DMA,
    ],
)(_sc_body)


def _tc_pre_body(obj_ref, pbox_ref, tbox_ref, out_ref):
    x = obj_ref[...]
    col = lax.broadcasted_iota(jnp.int32, (_B, _N), 1)
    t = (col < _M).astype(jnp.float32)
    bce = jnp.maximum(x, 0.0) - x * t + jnp.log1p(jnp.exp(-jnp.abs(x)))
    out_ref[0] = jnp.sum(bce) * (1.0 / (_B * _N))
    out_ref[1] = jnp.sum(jnp.abs(pbox_ref[...] - tbox_ref[...])) * (
        1.0 / (_B * _M * 4))


def _tc_fin_body(pre_ref, sumexp_ref, picked_ref, o0, o1, o2, o3):
    lse = jnp.log(sumexp_ref[...])
    class_loss = jnp.sum(lse - picked_ref[...]) * (1.0 / (_B * _M))
    obj_loss = pre_ref[0]
    box_loss = pre_ref[1]
    o0[0] = box_loss + obj_loss + class_loss
    o1[0] = box_loss
    o2[0] = obj_loss
    o3[0] = class_loss


def kernel(pred_boxes, pred_obj, pred_class, tgt_boxes, tgt_labels):
    labels = tgt_labels.astype(jnp.int32)
    cls_t = jnp.swapaxes(pred_class, 1, 2)
    sumexp, picked = _sc_call(cls_t, labels)

    pbt = jnp.swapaxes(pred_boxes, 1, 2)
    tbt = jnp.swapaxes(tgt_boxes, 1, 2)

    pre = pl.pallas_call(
        _tc_pre_body,
        out_shape=jax.ShapeDtypeStruct((2,), jnp.float32),
        grid=(1,),
        in_specs=[
            pl.BlockSpec((_B, _N), lambda i: (0, 0)),
            pl.BlockSpec((_B, 4, _M), lambda i: (0, 0, 0)),
            pl.BlockSpec((_B, 4, _M), lambda i: (0, 0, 0)),
        ],
        out_specs=pl.BlockSpec(memory_space=pltpu.SMEM),
    )(pred_obj, pbt, tbt)

    scalar = jax.ShapeDtypeStruct((1,), jnp.float32)
    smem = pl.BlockSpec(memory_space=pltpu.SMEM)
    o0, o1, o2, o3 = pl.pallas_call(
        _tc_fin_body,
        out_shape=[scalar, scalar, scalar, scalar],
        grid=(1,),
        in_specs=[
            smem,
            pl.BlockSpec((_B, _M), lambda i: (0, 0)),
            pl.BlockSpec((_B, _M), lambda i: (0, 0)),
        ],
        out_specs=[smem, smem, smem, smem],
    )(pre, sumexp, picked)
    return (o0.reshape(()), o1.reshape(()), o2.reshape(()), o3.reshape(()))
